# baseline (speedup 1.0000x reference)
import functools
import jax
import jax.numpy as jnp
from jax import lax
from jax.experimental import pallas as pl
from jax.experimental.pallas import tpu as pltpu
from jax.experimental.pallas import tpu_sc as plsc

N = 10000
E = 320000
F = 128

NC = 2
NS = 16
NW = NC * NS
EPW = E // NW
CHUNK = 128
EPT = 10240
EPAD = NW * EPT
NCHUNK = EPT // CHUNK
NPAIR = NCHUNK // 2
NPAD = 10240
RPT = NPAD // NS


def _mesh():
    return plsc.VectorSubcoreMesh(core_axis_name="c", subcore_axis_name="s")


def _deg_body(dst_hbm, ones_hbm, zeros_hbm, out_hbm, dst_v, ones_v, acc_sh):
    c = lax.axis_index("c")
    s = lax.axis_index("s")
    wid = s * NC + c
    pltpu.sync_copy(dst_hbm.at[wid], dst_v)
    pltpu.sync_copy(ones_hbm, ones_v)
    pltpu.sync_copy(zeros_hbm.at[pl.ds(s * RPT, RPT)],
                    acc_sh.at[pl.ds(s * RPT, RPT)])
    plsc.subcore_barrier()

    def body(j, carry):
        pltpu.sync_copy(ones_v, acc_sh.at[dst_v.at[j]], add=True)
        return carry

    lax.fori_loop(0, NCHUNK, body, 0)
    plsc.subcore_barrier()
    pltpu.sync_copy(acc_sh.at[pl.ds(s * RPT, RPT)],
                    out_hbm.at[c].at[pl.ds(s * RPT, RPT)])


def _degree_partials(dst_r, ones_in, zeros_in):
    fn = pl.kernel(
        _deg_body,
        out_type=jax.ShapeDtypeStruct((NC, NPAD), jnp.float32),
        mesh=_mesh(),
        scratch_types=[
            pltpu.VMEM((NCHUNK, CHUNK), jnp.int32),
            pltpu.VMEM((CHUNK,), jnp.float32),
            pltpu.VMEM_SHARED((NPAD,), jnp.float32),
        ],
    )
    return fn(dst_r, ones_in, zeros_in)


def _unpack_chunk(pk_v, j, sidx, didx):
    for k in range(CHUNK // 16):
        v = pk_v[j, pl.ds(k * 16, 16)]
        sidx[pl.ds(k * 16, 16)] = jnp.bitwise_and(v, jnp.int32(0xFFFF))
        didx[pl.ds(k * 16, 16)] = lax.shift_right_logical(v, jnp.int32(16))


def _edge_body(g_hbm, pk_hbm, zeros_hbm, out_hbm,
               pk_v, sidx0, didx0, sidx1, didx1, rows0, rows1, acc_sh,
               sem0, sem1):
    c = lax.axis_index("c")
    s = lax.axis_index("s")
    wid = s * NC + c
    pltpu.sync_copy(pk_hbm.at[wid], pk_v)
    pltpu.sync_copy(zeros_hbm.at[pl.ds(s * RPT, RPT)],
                    acc_sh.at[pl.ds(s * RPT, RPT)])
    plsc.subcore_barrier()

    _unpack_chunk(pk_v, 0, sidx0, didx0)
    pltpu.async_copy(g_hbm.at[sidx0], rows0, sem0)
    _unpack_chunk(pk_v, 1, sidx1, didx1)
    pltpu.async_copy(g_hbm.at[sidx1], rows1, sem1)

    def body(i, carry):
        j2 = 2 * i + 2
        pltpu.make_async_copy(g_hbm.at[sidx0], rows0, sem0).wait()
        pltpu.sync_copy(rows0, acc_sh.at[didx0], add=True)
        _unpack_chunk(pk_v, j2, sidx0, didx0)
        pltpu.async_copy(g_hbm.at[sidx0], rows0, sem0)
        pltpu.make_async_copy(g_hbm.at[sidx1], rows1, sem1).wait()
        pltpu.sync_copy(rows1, acc_sh.at[didx1], add=True)
        _unpack_chunk(pk_v, j2 + 1, sidx1, didx1)
        pltpu.async_copy(g_hbm.at[sidx1], rows1, sem1)
        return carry

    lax.fori_loop(0, NPAIR - 1, body, 0)
    pltpu.make_async_copy(g_hbm.at[sidx0], rows0, sem0).wait()
    pltpu.sync_copy(rows0, acc_sh.at[didx0], add=True)
    pltpu.make_async_copy(g_hbm.at[sidx1], rows1, sem1).wait()
    pltpu.sync_copy(rows1, acc_sh.at[didx1], add=True)
    plsc.subcore_barrier()
    pltpu.sync_copy(acc_sh.at[pl.ds(s * RPT, RPT)],
                    out_hbm.at[c].at[pl.ds(s * RPT, RPT)])


def _edge_partials(g, pk_r, zeros_in):
    fn = pl.kernel(
        _edge_body,
        out_type=jax.ShapeDtypeStruct((NC, NPAD, F), jnp.float32),
        mesh=_mesh(),
        scratch_types=[
            pltpu.VMEM((NCHUNK, CHUNK), jnp.int32),
            pltpu.VMEM((CHUNK,), jnp.int32),
            pltpu.VMEM((CHUNK,), jnp.int32),
            pltpu.VMEM((CHUNK,), jnp.int32),
            pltpu.VMEM((CHUNK,), jnp.int32),
            pltpu.VMEM((CHUNK, F), jnp.float32),
            pltpu.VMEM((CHUNK, F), jnp.float32),
            pltpu.VMEM_SHARED((NPAD, F), jnp.float32),
            pltpu.SemaphoreType.DMA,
            pltpu.SemaphoreType.DMA,
        ],
    )
    return fn(g, pk_r, zeros_in)


BR = 1000


def _prep_body(degp_ref, x_ref, w_ref, g_ref, dinvb_ref):
    deg = degp_ref[:, 0] + degp_ref[:, 1] + 1.0
    dinv = lax.rsqrt(deg)
    h = jnp.dot(x_ref[...], w_ref[...], preferred_element_type=jnp.float32)
    g_ref[...] = h * dinv[:, None]
    dinvb_ref[...] = jnp.broadcast_to(dinv[:, None], (BR, F))


def _tc_prep(degp, x, w1):
    return pl.pallas_call(
        _prep_body,
        grid=(N // BR,),
        in_specs=[
            pl.BlockSpec((BR, NC), lambda i: (i, 0)),
            pl.BlockSpec((BR, F), lambda i: (i, 0)),
            pl.BlockSpec((F, F), lambda i: (0, 0)),
        ],
        out_specs=[
            pl.BlockSpec((BR, F), lambda i: (i, 0)),
            pl.BlockSpec((BR, F), lambda i: (i, 0)),
        ],
        out_shape=[
            jax.ShapeDtypeStruct((N, F), jnp.float32),
            jax.ShapeDtypeStruct((N, F), jnp.float32),
        ],
    )(degp, x, w1)


def _mid_body(p_ref, g_ref, dinvb_ref, b_ref, w_ref, g2_ref):
    psum = p_ref[0] + p_ref[1] + g_ref[...]
    t = jnp.maximum(dinvb_ref[...] * psum + b_ref[...], 0.0)
    h = jnp.dot(t, w_ref[...], preferred_element_type=jnp.float32)
    g2_ref[...] = h * dinvb_ref[...]


def _tc_mid(p, g, dinvb, b1, w2):
    return pl.pallas_call(
        _mid_body,
        grid=(N // BR,),
        in_specs=[
            pl.BlockSpec((NC, BR, F), lambda i: (0, i, 0)),
            pl.BlockSpec((BR, F), lambda i: (i, 0)),
            pl.BlockSpec((BR, F), lambda i: (i, 0)),
            pl.BlockSpec((1, F), lambda i: (0, 0)),
            pl.BlockSpec((F, F), lambda i: (0, 0)),
        ],
        out_specs=pl.BlockSpec((BR, F), lambda i: (i, 0)),
        out_shape=jax.ShapeDtypeStruct((N, F), jnp.float32),
    )(p, g, dinvb, b1, w2)


def _fin_body(p_ref, g_ref, dinvb_ref, b_ref, out_ref):
    psum = p_ref[0] + p_ref[1] + g_ref[...]
    out_ref[...] = jnp.maximum(dinvb_ref[...] * psum + b_ref[...], 0.0)


def _tc_fin(p, g, dinvb, b2):
    return pl.pallas_call(
        _fin_body,
        grid=(N // BR,),
        in_specs=[
            pl.BlockSpec((NC, BR, F), lambda i: (0, i, 0)),
            pl.BlockSpec((BR, F), lambda i: (i, 0)),
            pl.BlockSpec((BR, F), lambda i: (i, 0)),
            pl.BlockSpec((1, F), lambda i: (0, 0)),
        ],
        out_specs=pl.BlockSpec((BR, F), lambda i: (i, 0)),
        out_shape=jax.ShapeDtypeStruct((N, F), jnp.float32),
    )(p, g, dinvb, b2)


def kernel(x, edge_index, W1, b1, W2, b2):
    npadedge = EPAD - E
    src_p = jnp.concatenate(
        [edge_index[0], jnp.zeros((npadedge,), jnp.int32)])
    dst_p = jnp.concatenate(
        [edge_index[1], jnp.full((npadedge,), N, jnp.int32)])
    pk_r = (src_p | (dst_p << 16)).reshape(NW, NCHUNK, CHUNK)
    dst_r = dst_p.reshape(NW, NCHUNK, CHUNK)
    ones_in = jnp.ones((CHUNK,), jnp.float32)
    zeros16 = jnp.zeros((NPAD,), jnp.float32)
    zerosF = jnp.zeros((NPAD, F), jnp.float32)
    b1r = b1.reshape(1, F)
    b2r = b2.reshape(1, F)

    degp = _degree_partials(dst_r, ones_in, zeros16)
    g1, dinvb = _tc_prep(jnp.swapaxes(degp, 0, 1), x, W1)
    p1 = _edge_partials(g1, pk_r, zerosF)
    g2 = _tc_mid(p1, g1, dinvb, b1r, W2)
    p2 = _edge_partials(g2, pk_r, zerosF)
    return _tc_fin(p2, g2, dinvb, b2r)


# speedup vs baseline: 1.0224x; 1.0224x over previous
import functools
import jax
import jax.numpy as jnp
from jax import lax
from jax.experimental import pallas as pl
from jax.experimental.pallas import tpu as pltpu
from jax.experimental.pallas import tpu_sc as plsc

N = 10000
E = 320000
F = 128

NC = 2
NS = 16
NW = NC * NS
EPW = E // NW
CHUNK = 128
EPT = 10240
EPAD = NW * EPT
NCHUNK = EPT // CHUNK
NPAIR = NCHUNK // 2
NPAD = 10240
RPT = NPAD // NS


def _mesh():
    return plsc.VectorSubcoreMesh(core_axis_name="c", subcore_axis_name="s")


def _deg_body(dst_hbm, ones_hbm, zeros_hbm, out_hbm, dst_v, ones_v, acc_sh):
    c = lax.axis_index("c")
    s = lax.axis_index("s")
    wid = s * NC + c
    pltpu.sync_copy(dst_hbm.at[wid], dst_v)
    pltpu.sync_copy(ones_hbm, ones_v)
    pltpu.sync_copy(zeros_hbm.at[pl.ds(s * RPT, RPT)],
                    acc_sh.at[pl.ds(s * RPT, RPT)])
    plsc.subcore_barrier()

    def body(j, carry):
        pltpu.sync_copy(ones_v, acc_sh.at[dst_v.at[j]], add=True)
        return carry

    lax.fori_loop(0, NCHUNK, body, 0)
    plsc.subcore_barrier()
    pltpu.sync_copy(acc_sh.at[pl.ds(s * RPT, RPT)],
                    out_hbm.at[c].at[pl.ds(s * RPT, RPT)])


def _degree_partials(dst_r, ones_in, zeros_in):
    fn = pl.kernel(
        _deg_body,
        out_type=jax.ShapeDtypeStruct((NC, NPAD), jnp.float32),
        mesh=_mesh(),
        scratch_types=[
            pltpu.VMEM((NCHUNK, CHUNK), jnp.int32),
            pltpu.VMEM((CHUNK,), jnp.float32),
            pltpu.VMEM_SHARED((NPAD,), jnp.float32),
        ],
    )
    return fn(dst_r, ones_in, zeros_in)


def _unpack_chunk(pk_v, j, sidx, didx):
    for k in range(CHUNK // 16):
        v = pk_v[j, pl.ds(k * 16, 16)]
        sidx[pl.ds(k * 16, 16)] = jnp.bitwise_and(v, jnp.int32(0xFFFF))
        didx[pl.ds(k * 16, 16)] = lax.shift_right_logical(v, jnp.int32(16))


def _edge_body(g_hbm, pk_hbm, zeros_hbm, out_hbm,
               pk_v, sidx0, didx0, sidx1, didx1, rows0, rows1, acc_sh,
               sem0, sem1):
    c = lax.axis_index("c")
    s = lax.axis_index("s")
    wid = s * NC + c
    pltpu.sync_copy(pk_hbm.at[wid], pk_v)
    pltpu.sync_copy(zeros_hbm.at[pl.ds(s * RPT, RPT)],
                    acc_sh.at[pl.ds(s * RPT, RPT)])
    plsc.subcore_barrier()

    _unpack_chunk(pk_v, 0, sidx0, didx0)
    pltpu.async_copy(g_hbm.at[sidx0], rows0, sem0)
    _unpack_chunk(pk_v, 1, sidx1, didx1)
    pltpu.async_copy(g_hbm.at[sidx1], rows1, sem1)

    def body(i, carry):
        j2 = 2 * i + 2
        pltpu.make_async_copy(g_hbm.at[sidx0], rows0, sem0).wait()
        pltpu.sync_copy(rows0, acc_sh.at[didx0], add=True)
        _unpack_chunk(pk_v, j2, sidx0, didx0)
        pltpu.async_copy(g_hbm.at[sidx0], rows0, sem0)
        pltpu.make_async_copy(g_hbm.at[sidx1], rows1, sem1).wait()
        pltpu.sync_copy(rows1, acc_sh.at[didx1], add=True)
        _unpack_chunk(pk_v, j2 + 1, sidx1, didx1)
        pltpu.async_copy(g_hbm.at[sidx1], rows1, sem1)
        return carry

    lax.fori_loop(0, NPAIR - 1, body, 0)
    pltpu.make_async_copy(g_hbm.at[sidx0], rows0, sem0).wait()
    pltpu.sync_copy(rows0, acc_sh.at[didx0], add=True)
    pltpu.make_async_copy(g_hbm.at[sidx1], rows1, sem1).wait()
    pltpu.sync_copy(rows1, acc_sh.at[didx1], add=True)
    plsc.subcore_barrier()
    pltpu.sync_copy(acc_sh.at[pl.ds(s * RPT, RPT)],
                    out_hbm.at[c].at[pl.ds(s * RPT, RPT)])


def _edge_partials(g, pk_r, zeros_in):
    fn = pl.kernel(
        _edge_body,
        out_type=jax.ShapeDtypeStruct((NC, NPAD, F), jnp.float32),
        mesh=_mesh(),
        scratch_types=[
            pltpu.VMEM((NCHUNK, CHUNK), jnp.int32),
            pltpu.VMEM((CHUNK,), jnp.int32),
            pltpu.VMEM((CHUNK,), jnp.int32),
            pltpu.VMEM((CHUNK,), jnp.int32),
            pltpu.VMEM((CHUNK,), jnp.int32),
            pltpu.VMEM((CHUNK, F), jnp.float32),
            pltpu.VMEM((CHUNK, F), jnp.float32),
            pltpu.VMEM_SHARED((NPAD, F), jnp.float32),
            pltpu.SemaphoreType.DMA,
            pltpu.SemaphoreType.DMA,
        ],
    )
    return fn(g, pk_r, zeros_in)


BR = 1000


def _prep_body(degp_ref, x_ref, w_ref, g_ref, dinvb_ref):
    deg = degp_ref[:, 0] + degp_ref[:, 1] + 1.0
    dinv = lax.rsqrt(deg)
    h = jnp.dot(x_ref[...], w_ref[...], preferred_element_type=jnp.float32)
    g_ref[...] = h * dinv[:, None]
    dinvb_ref[...] = jnp.broadcast_to(dinv[:, None], (BR, F))


def _tc_prep(degp, x, w1):
    return pl.pallas_call(
        _prep_body,
        grid=(N // BR,),
        in_specs=[
            pl.BlockSpec((BR, NC), lambda i: (i, 0)),
            pl.BlockSpec((BR, F), lambda i: (i, 0)),
            pl.BlockSpec((F, F), lambda i: (0, 0)),
        ],
        out_specs=[
            pl.BlockSpec((BR, F), lambda i: (i, 0)),
            pl.BlockSpec((BR, F), lambda i: (i, 0)),
        ],
        out_shape=[
            jax.ShapeDtypeStruct((N, F), jnp.float32),
            jax.ShapeDtypeStruct((N, F), jnp.float32),
        ],
    )(degp, x, w1)


def _mid_body(p_ref, g_ref, dinvb_ref, b_ref, w_ref, g2_ref):
    psum = p_ref[0] + p_ref[1] + g_ref[...]
    t = jnp.maximum(dinvb_ref[...] * psum + b_ref[...], 0.0)
    h = jnp.dot(t, w_ref[...], preferred_element_type=jnp.float32)
    g2_ref[...] = h * dinvb_ref[...]


def _tc_mid(p, g, dinvb, b1, w2):
    return pl.pallas_call(
        _mid_body,
        grid=(N // BR,),
        in_specs=[
            pl.BlockSpec((NC, BR, F), lambda i: (0, i, 0)),
            pl.BlockSpec((BR, F), lambda i: (i, 0)),
            pl.BlockSpec((BR, F), lambda i: (i, 0)),
            pl.BlockSpec((1, F), lambda i: (0, 0)),
            pl.BlockSpec((F, F), lambda i: (0, 0)),
        ],
        out_specs=pl.BlockSpec((BR, F), lambda i: (i, 0)),
        out_shape=jax.ShapeDtypeStruct((N, F), jnp.float32),
    )(p, g, dinvb, b1, w2)


def _fin_body(p_ref, g_ref, dinvb_ref, b_ref, out_ref):
    psum = p_ref[0] + p_ref[1] + g_ref[...]
    out_ref[...] = jnp.maximum(dinvb_ref[...] * psum + b_ref[...], 0.0)


def _tc_fin(p, g, dinvb, b2):
    return pl.pallas_call(
        _fin_body,
        grid=(N // BR,),
        in_specs=[
            pl.BlockSpec((NC, BR, F), lambda i: (0, i, 0)),
            pl.BlockSpec((BR, F), lambda i: (i, 0)),
            pl.BlockSpec((BR, F), lambda i: (i, 0)),
            pl.BlockSpec((1, F), lambda i: (0, 0)),
        ],
        out_specs=pl.BlockSpec((BR, F), lambda i: (i, 0)),
        out_shape=jax.ShapeDtypeStruct((N, F), jnp.float32),
    )(p, g, dinvb, b2)


def kernel(x, edge_index, W1, b1, W2, b2):
    npadedge = EPAD - E
    src_p = jnp.concatenate(
        [edge_index[0], jnp.zeros((npadedge,), jnp.int32)])
    pad_dst = N + (jnp.arange(npadedge, dtype=jnp.int32) % (NPAD - N))
    dst_p = jnp.concatenate([edge_index[1], pad_dst])
    pk_r = (src_p | (dst_p << 16)).reshape(NW, NCHUNK, CHUNK)
    dst_r = dst_p.reshape(NW, NCHUNK, CHUNK)
    ones_in = jnp.ones((CHUNK,), jnp.float32)
    zeros16 = jnp.zeros((NPAD,), jnp.float32)
    zerosF = jnp.zeros((NPAD, F), jnp.float32)
    b1r = b1.reshape(1, F)
    b2r = b2.reshape(1, F)

    degp = _degree_partials(dst_r, ones_in, zeros16)
    g1, dinvb = _tc_prep(jnp.swapaxes(degp, 0, 1), x, W1)
    p1 = _edge_partials(g1, pk_r, zerosF)
    g2 = _tc_mid(p1, g1, dinvb, b1r, W2)
    p2 = _edge_partials(g2, pk_r, zerosF)
    return _tc_fin(p2, g2, dinvb, b2r)


# speedup vs baseline: 3.4478x; 3.3724x over previous
import functools
import jax
import jax.numpy as jnp
from jax import lax
from jax.experimental import pallas as pl
from jax.experimental.pallas import tpu as pltpu
from jax.experimental.pallas import tpu_sc as plsc

N = 10000
E = 320000
F = 128

NC = 2
NS = 16
NW = NC * NS
EPW = E // NW
CHUNK = 128
EPT = 10240
EPAD = NW * EPT
NCHUNK = EPT // CHUNK
NPAIR = NCHUNK // 2
NPAD = 10240
RPT = NPAD // NS


def _mesh():
    return plsc.VectorSubcoreMesh(core_axis_name="c", subcore_axis_name="s")


def _deg_body(dst_hbm, ones_hbm, zeros_hbm, out_hbm, dst_v, ones_v, acc_sh):
    c = lax.axis_index("c")
    s = lax.axis_index("s")
    wid = s * NC + c
    pltpu.sync_copy(dst_hbm.at[wid], dst_v)
    pltpu.sync_copy(ones_hbm, ones_v)
    pltpu.sync_copy(zeros_hbm.at[pl.ds(s * RPT, RPT)],
                    acc_sh.at[pl.ds(s * RPT, RPT)])
    plsc.subcore_barrier()

    def body(j, carry):
        pltpu.sync_copy(ones_v, acc_sh.at[dst_v.at[j]], add=True)
        return carry

    lax.fori_loop(0, NCHUNK, body, 0)
    plsc.subcore_barrier()
    pltpu.sync_copy(acc_sh.at[pl.ds(s * RPT, RPT)],
                    out_hbm.at[c].at[pl.ds(s * RPT, RPT)])


def _degree_partials(dst_r, ones_in, zeros_in):
    fn = pl.kernel(
        _deg_body,
        out_type=jax.ShapeDtypeStruct((NC, NPAD), jnp.float32),
        mesh=_mesh(),
        scratch_types=[
            pltpu.VMEM((NCHUNK, CHUNK), jnp.int32),
            pltpu.VMEM((CHUNK,), jnp.float32),
            pltpu.VMEM_SHARED((NPAD,), jnp.float32),
        ],
    )
    return fn(dst_r, ones_in, zeros_in)


def _unpack_chunk(pk_v, j, sidx, didx):
    for k in range(CHUNK // 16):
        v = pk_v[j, pl.ds(k * 16, 16)]
        sidx[pl.ds(k * 16, 16)] = jnp.bitwise_and(v, jnp.int32(0xFFFF))
        didx[pl.ds(k * 16, 16)] = lax.shift_right_logical(v, jnp.int32(16))


def _edge_body(g_hbm, pk_hbm, zeros_hbm, out_hbm,
               pk_v, sidx0, didx0, sidx1, didx1, rows0, rows1, acc_sh,
               sem0, sem1):
    c = lax.axis_index("c")
    s = lax.axis_index("s")
    wid = s * NC + c
    pltpu.sync_copy(pk_hbm.at[wid], pk_v)
    pltpu.sync_copy(zeros_hbm.at[pl.ds(s * RPT, RPT)],
                    acc_sh.at[pl.ds(s * RPT, RPT)])
    plsc.subcore_barrier()

    _unpack_chunk(pk_v, 0, sidx0, didx0)
    pltpu.async_copy(g_hbm.at[sidx0], rows0, sem0)
    _unpack_chunk(pk_v, 1, sidx1, didx1)
    pltpu.async_copy(g_hbm.at[sidx1], rows1, sem1)

    def body(i, carry):
        j2 = 2 * i + 2
        pltpu.make_async_copy(g_hbm.at[sidx0], rows0, sem0).wait()
        pltpu.sync_copy(rows0, acc_sh.at[didx0], add=True)
        _unpack_chunk(pk_v, j2, sidx0, didx0)
        pltpu.async_copy(g_hbm.at[sidx0], rows0, sem0)
        pltpu.make_async_copy(g_hbm.at[sidx1], rows1, sem1).wait()
        pltpu.sync_copy(rows1, acc_sh.at[didx1], add=True)
        _unpack_chunk(pk_v, j2 + 1, sidx1, didx1)
        pltpu.async_copy(g_hbm.at[sidx1], rows1, sem1)
        return carry

    lax.fori_loop(0, NPAIR - 1, body, 0)
    pltpu.make_async_copy(g_hbm.at[sidx0], rows0, sem0).wait()
    pltpu.sync_copy(rows0, acc_sh.at[didx0], add=True)
    pltpu.make_async_copy(g_hbm.at[sidx1], rows1, sem1).wait()
    pltpu.sync_copy(rows1, acc_sh.at[didx1], add=True)
    plsc.subcore_barrier()
    pltpu.sync_copy(acc_sh.at[pl.ds(s * RPT, RPT)],
                    out_hbm.at[c].at[pl.ds(s * RPT, RPT)])


def _edge_partials(g, pk_r, zeros_in):
    fn = pl.kernel(
        _edge_body,
        out_type=jax.ShapeDtypeStruct((NC, NPAD, F), jnp.float32),
        mesh=_mesh(),
        scratch_types=[
            pltpu.VMEM((NCHUNK, CHUNK), jnp.int32),
            pltpu.VMEM((CHUNK,), jnp.int32),
            pltpu.VMEM((CHUNK,), jnp.int32),
            pltpu.VMEM((CHUNK,), jnp.int32),
            pltpu.VMEM((CHUNK,), jnp.int32),
            pltpu.VMEM((CHUNK, F), jnp.float32),
            pltpu.VMEM((CHUNK, F), jnp.float32),
            pltpu.VMEM_SHARED((NPAD, F), jnp.float32),
            pltpu.SemaphoreType.DMA,
            pltpu.SemaphoreType.DMA,
        ],
    )
    return fn(g, pk_r, zeros_in)


BR = 1000


def _prep_body(degp_ref, x_ref, w_ref, g_ref, dinvb_ref):
    deg = degp_ref[:, 0] + degp_ref[:, 1] + 1.0
    dinv = lax.rsqrt(deg)
    h = jnp.dot(x_ref[...], w_ref[...], preferred_element_type=jnp.float32)
    g_ref[...] = h * dinv[:, None]
    dinvb_ref[...] = jnp.broadcast_to(dinv[:, None], (BR, F))


def _tc_prep(degp, x, w1):
    return pl.pallas_call(
        _prep_body,
        grid=(N // BR,),
        in_specs=[
            pl.BlockSpec((BR, NC), lambda i: (i, 0)),
            pl.BlockSpec((BR, F), lambda i: (i, 0)),
            pl.BlockSpec((F, F), lambda i: (0, 0)),
        ],
        out_specs=[
            pl.BlockSpec((BR, F), lambda i: (i, 0)),
            pl.BlockSpec((BR, F), lambda i: (i, 0)),
        ],
        out_shape=[
            jax.ShapeDtypeStruct((N, F), jnp.float32),
            jax.ShapeDtypeStruct((N, F), jnp.float32),
        ],
    )(degp, x, w1)


def _mid_body(p_ref, g_ref, dinvb_ref, b_ref, w_ref, g2_ref):
    psum = p_ref[0] + p_ref[1] + g_ref[...]
    t = jnp.maximum(dinvb_ref[...] * psum + b_ref[...], 0.0)
    h = jnp.dot(t, w_ref[...], preferred_element_type=jnp.float32)
    g2_ref[...] = h * dinvb_ref[...]


def _tc_mid(p, g, dinvb, b1, w2):
    return pl.pallas_call(
        _mid_body,
        grid=(N // BR,),
        in_specs=[
            pl.BlockSpec((NC, BR, F), lambda i: (0, i, 0)),
            pl.BlockSpec((BR, F), lambda i: (i, 0)),
            pl.BlockSpec((BR, F), lambda i: (i, 0)),
            pl.BlockSpec((1, F), lambda i: (0, 0)),
            pl.BlockSpec((F, F), lambda i: (0, 0)),
        ],
        out_specs=pl.BlockSpec((BR, F), lambda i: (i, 0)),
        out_shape=jax.ShapeDtypeStruct((N, F), jnp.float32),
    )(p, g, dinvb, b1, w2)


def _fin_body(p_ref, g_ref, dinvb_ref, b_ref, out_ref):
    psum = p_ref[0] + p_ref[1] + g_ref[...]
    out_ref[...] = jnp.maximum(dinvb_ref[...] * psum + b_ref[...], 0.0)


def _tc_fin(p, g, dinvb, b2):
    return pl.pallas_call(
        _fin_body,
        grid=(N // BR,),
        in_specs=[
            pl.BlockSpec((NC, BR, F), lambda i: (0, i, 0)),
            pl.BlockSpec((BR, F), lambda i: (i, 0)),
            pl.BlockSpec((BR, F), lambda i: (i, 0)),
            pl.BlockSpec((1, F), lambda i: (0, 0)),
        ],
        out_specs=pl.BlockSpec((BR, F), lambda i: (i, 0)),
        out_shape=jax.ShapeDtypeStruct((N, F), jnp.float32),
    )(p, g, dinvb, b2)


def kernel(x, edge_index, W1, b1, W2, b2):
    npadedge = EPAD - E
    pad_src = jnp.arange(npadedge, dtype=jnp.int32) % N
    src_p = jnp.concatenate([edge_index[0], pad_src])
    pad_dst = N + (jnp.arange(npadedge, dtype=jnp.int32) % (NPAD - N))
    dst_p = jnp.concatenate([edge_index[1], pad_dst])
    pk_r = (src_p | (dst_p << 16)).reshape(NW, NCHUNK, CHUNK)
    dst_r = dst_p.reshape(NW, NCHUNK, CHUNK)
    ones_in = jnp.ones((CHUNK,), jnp.float32)
    zeros16 = jnp.zeros((NPAD,), jnp.float32)
    zerosF = jnp.zeros((NPAD, F), jnp.float32)
    b1r = b1.reshape(1, F)
    b2r = b2.reshape(1, F)

    degp = _degree_partials(dst_r, ones_in, zeros16)
    g1, dinvb = _tc_prep(jnp.swapaxes(degp, 0, 1), x, W1)
    p1 = _edge_partials(g1, pk_r, zerosF)
    g2 = _tc_mid(p1, g1, dinvb, b1r, W2)
    p2 = _edge_partials(g2, pk_r, zerosF)
    return _tc_fin(p2, g2, dinvb, b2r)
